# baseline (speedup 1.0000x reference)
import functools

import jax
import jax.numpy as jnp
from jax import lax
from jax.experimental import pallas as pl
from jax.experimental.pallas import tpu as pltpu
from jax.experimental.pallas import tpu_sc as plsc

_BATCH = 16
_H = 64
_W = 64


def kernel(batch, length_h, length_w, embeddings_tablex, embeddings_tabley):
    n, feat = embeddings_tablex.shape
    cat = jnp.concatenate(
        [embeddings_tablex, jnp.flip(embeddings_tablex, axis=0)], axis=1
    ).reshape(-1)

    info = plsc.get_sparse_core_info()
    nw = info.num_cores * info.num_subcores
    jobs_per_w = (_BATCH * _H) // nw
    i_span = _H // (nw // _BATCH)

    mesh = plsc.VectorSubcoreMesh(core_axis_name="c", subcore_axis_name="s")

    slab = _W * 2 * feat
    img = _H * _W * 2 * feat
    ns = info.num_subcores
    half = img // 2

    @functools.partial(
        pl.kernel,
        out_type=jax.ShapeDtypeStruct((_BATCH * _H * _W * 2 * feat,), jnp.float32),
        mesh=mesh,
        scratch_types=[
            pltpu.VMEM((n * 2 * feat,), jnp.float32),
            pltpu.VMEM_SHARED((img,), jnp.float32),
            pltpu.SemaphoreType.DMA,
        ],
    )
    def relpos(cat_hbm, out_hbm, cat_v, img_sh, sem):
        c = lax.axis_index("c")
        s = lax.axis_index("s")
        pltpu.sync_copy(cat_hbm, cat_v)
        for t in range(_H // ns):
            i = s * (_H // ns) + t
            pltpu.sync_copy(
                cat_v.at[pl.ds((_H - i) * 2 * feat, slab)],
                img_sh.at[pl.ds(i * slab, slab)],
            )
        plsc.subcore_barrier()
        b = c * (_BATCH // 2) + s // 2
        dst = b * img + (s % 2) * half
        pltpu.async_copy(
            img_sh.at[pl.ds((s % 2) * half, half)],
            out_hbm.at[pl.ds(dst, half)],
            sem,
        ).wait()

    out = relpos(cat)
    return out.reshape(_BATCH, _H * _W, 2 * feat)


# speedup vs baseline: 3.1559x; 3.1559x over previous
import functools

import jax
import jax.numpy as jnp
from jax import lax
from jax.experimental import pallas as pl
from jax.experimental.pallas import tpu as pltpu
from jax.experimental.pallas import tpu_sc as plsc

_BATCH = 16
_H = 64
_W = 64


def kernel(batch, length_h, length_w, embeddings_tablex, embeddings_tabley):
    n, feat = embeddings_tablex.shape
    cat = jnp.concatenate(
        [embeddings_tablex, jnp.flip(embeddings_tablex, axis=0)], axis=1
    )
    catp = jnp.pad(cat, ((0, 15), (0, 0)))
    shift8 = jnp.stack([catp[k : k + 136] for k in range(8)])

    info = plsc.get_sparse_core_info()
    nw = info.num_cores * info.num_subcores

    mesh = plsc.VectorSubcoreMesh(core_axis_name="c", subcore_axis_name="s")

    @functools.partial(
        pl.kernel,
        out_type=jax.ShapeDtypeStruct((_BATCH, _H * _W, 2 * feat), jnp.float32),
        mesh=mesh,
        scratch_types=[
            pltpu.VMEM((136, 2 * feat), jnp.float32),
            pltpu.SemaphoreType.DMA,
        ],
    )
    def relpos(shift_hbm, out_hbm, tab_v, sem):
        wid = lax.axis_index("s") * info.num_cores + lax.axis_index("c")
        r = wid % 8
        k = (8 - r) % 8
        b0 = (wid // 8) * 4
        pltpu.sync_copy(shift_hbm.at[k], tab_v)
        copies = []
        for m in range(8):
            i = r + 8 * m
            a = pl.multiple_of((_H - k) - i, 8)
            for t in range(4):
                copies.append(
                    pltpu.async_copy(
                        tab_v.at[pl.ds(a, _W), :],
                        out_hbm.at[b0 + t, pl.ds(i * _W, _W), :],
                        sem,
                    )
                )
        for c in copies:
            c.wait()

    return relpos(shift8)
